# baseline (speedup 1.0000x reference)
import functools

import jax
import jax.numpy as jnp
from jax import lax
from jax.experimental import pallas as pl
from jax.experimental.pallas import tpu as pltpu
from jax.experimental.pallas import tpu_sc as plsc

BATCH = 16384
N_CAT = 26
N_CONT = 26
VOCAB = 100000
EMB_DIM = 16
OUT_D = N_CAT * EMB_DIM + N_CONT

NW = 32
N_ROWS = N_CAT * EMB_DIM
ROWS_PER_W = N_ROWS // NW
HALF = 49920
HIGH = VOCAB - HALF
ICH = 4096
N_ICH = BATCH // ICH

_mesh = plsc.VectorSubcoreMesh(core_axis_name="c", subcore_axis_name="s")


@functools.partial(
    pl.kernel,
    mesh=_mesh,
    out_type=jax.ShapeDtypeStruct((OUT_D, BATCH), jnp.float32),
    scratch_types=[
        pltpu.VMEM((HALF,), jnp.float32),
        pltpu.VMEM((HIGH,), jnp.float32),
        pltpu.VMEM((ICH,), jnp.int32),
        pltpu.VMEM((ICH,), jnp.int32),
        pltpu.VMEM((BATCH,), jnp.float32),
        pltpu.SemaphoreType.DMA,
        pltpu.SemaphoreType.DMA,
        pltpu.SemaphoreType.DMA,
        pltpu.SemaphoreType.DMA,
    ],
    compiler_params=pltpu.CompilerParams(needs_layout_passes=False),
)
def _emb_kernel(tab_hbm, xt_hbm, out_hbm, blo, bhi, ia, ib, out_v,
                slo, shi, sa, sb):
    w = lax.axis_index("s") * 2 + lax.axis_index("c")
    ibufs = (ia, ib)
    isems = (sa, sb)
    lanes = lax.iota(jnp.int32, 16)

    def fe(rr):
        return rr // EMB_DIM, rr % EMB_DIM

    r0 = w * ROWS_PER_W
    f0, e0 = fe(r0)
    pltpu.async_copy(tab_hbm.at[f0, e0].at[pl.ds(0, HALF)], blo, slo)
    pltpu.async_copy(xt_hbm.at[f0].at[pl.ds(0, ICH)], ia, sa)

    def _task(k, _):
        r = w * ROWS_PER_W + k
        f, e = fe(r)
        row = tab_hbm.at[f, e]
        pltpu.async_copy(row.at[pl.ds(HALF, HIGH)], bhi, shi)
        pltpu.make_async_copy(row.at[pl.ds(0, HALF)], blo, slo).wait()

        for c8 in range(2 * N_ICH):
            jc = c8 % N_ICH
            par = c8 % 2
            if c8 < 2 * N_ICH - 1:
                njc = (c8 + 1) % N_ICH
                pltpu.async_copy(
                    xt_hbm.at[f].at[pl.ds(njc * ICH, ICH)],
                    ibufs[1 - par], isems[1 - par])
            else:

                @pl.when(k < ROWS_PER_W - 1)
                def _():
                    fn, _en = fe(r + 1)
                    pltpu.async_copy(
                        xt_hbm.at[fn].at[pl.ds(0, ICH)], ibufs[0], isems[0])

            if c8 == N_ICH:
                @pl.when(k < ROWS_PER_W - 1)
                def _():
                    fn, en = fe(r + 1)
                    pltpu.async_copy(
                        tab_hbm.at[fn, en].at[pl.ds(0, HALF)], blo, slo)

                pltpu.make_async_copy(row.at[pl.ds(HALF, HIGH)],
                                      bhi, shi).wait()

            idx_v = ibufs[par]
            pltpu.make_async_copy(xt_hbm.at[f].at[pl.ds(jc * ICH, ICH)],
                                  idx_v, isems[par]).wait()

            if c8 < N_ICH:

                @plsc.parallel_loop(0, ICH // 16, unroll=16)
                def _groups(i):
                    g = idx_v[pl.ds(i * 16, 16)]
                    m = g < HALF
                    out_v[pl.ds(jc * ICH + i * 16, 16)] = (
                        plsc.load_gather(blo, [g], mask=m))

            else:

                @plsc.parallel_loop(0, ICH // 16, unroll=16)
                def _groups(i):
                    g = idx_v[pl.ds(i * 16, 16)]
                    m = g >= HALF
                    gh = plsc.load_gather(bhi, [g - HALF], mask=m)
                    plsc.store_scatter(
                        out_v, [lanes + (jc * ICH + i * 16)], gh, mask=m)

        pltpu.sync_copy(out_v, out_hbm.at[r])
        return 0

    lax.fori_loop(0, ROWS_PER_W, _task, 0)

    @pl.when(w < N_CONT)
    def _cont():
        def _cchunk(j, _):
            pltpu.sync_copy(xt_hbm.at[N_CAT + w].at[pl.ds(j * ICH, ICH)], ia)

            @plsc.parallel_loop(0, ICH // 16, unroll=16)
            def _cgroups(i):
                out_v[pl.ds(j * ICH + i * 16, 16)] = (
                    ia[pl.ds(i * 16, 16)].astype(jnp.float32))

            return 0

        lax.fori_loop(0, N_ICH, _cchunk, 0)
        pltpu.sync_copy(out_v, out_hbm.at[N_ROWS + w])


def kernel(x, tables):
    tab_t = jnp.transpose(tables, (0, 2, 1))
    xt = jnp.transpose(x)
    out_t = _emb_kernel(tab_t, xt)
    return jnp.transpose(out_t)


# speedup vs baseline: 1.0002x; 1.0002x over previous
import functools

import jax
import jax.numpy as jnp
from jax import lax
from jax.experimental import pallas as pl
from jax.experimental.pallas import tpu as pltpu
from jax.experimental.pallas import tpu_sc as plsc

BATCH = 16384
N_CAT = 26
N_CONT = 26
VOCAB = 100000
EMB_DIM = 16
OUT_D = N_CAT * EMB_DIM + N_CONT

NW = 32
N_ROWS = N_CAT * EMB_DIM
ROWS_PER_W = N_ROWS // NW
HALF = 49920
HIGH = VOCAB - HALF
ICH = 4096
N_ICH = BATCH // ICH

_mesh = plsc.VectorSubcoreMesh(core_axis_name="c", subcore_axis_name="s")


@functools.partial(
    pl.kernel,
    mesh=_mesh,
    out_type=jax.ShapeDtypeStruct((OUT_D, BATCH), jnp.float32),
    scratch_types=[
        pltpu.VMEM((HALF,), jnp.float32),
        pltpu.VMEM((HIGH,), jnp.float32),
        pltpu.VMEM((ICH,), jnp.int32),
        pltpu.VMEM((ICH,), jnp.int32),
        pltpu.VMEM((BATCH,), jnp.float32),
        pltpu.SemaphoreType.DMA,
        pltpu.SemaphoreType.DMA,
        pltpu.SemaphoreType.DMA,
        pltpu.SemaphoreType.DMA,
        pltpu.SemaphoreType.DMA,
    ],
    compiler_params=pltpu.CompilerParams(needs_layout_passes=False),
)
def _emb_kernel(tab_hbm, xt_hbm, out_hbm, blo, bhi, ia, ib, out_v,
                slo, shi, sa, sb, so):
    w = lax.axis_index("s") * 2 + lax.axis_index("c")
    ibufs = (ia, ib)
    isems = (sa, sb)
    lanes = lax.iota(jnp.int32, 16)

    def fe(rr):
        return rr // EMB_DIM, rr % EMB_DIM

    r0 = w * ROWS_PER_W
    f0, e0 = fe(r0)
    pltpu.async_copy(tab_hbm.at[f0, e0].at[pl.ds(0, HALF)], blo, slo)
    pltpu.async_copy(xt_hbm.at[f0].at[pl.ds(0, ICH)], ia, sa)

    def _task(k, _):
        r = w * ROWS_PER_W + k
        f, e = fe(r)
        row = tab_hbm.at[f, e]
        pltpu.async_copy(row.at[pl.ds(HALF, HIGH)], bhi, shi)
        pltpu.make_async_copy(row.at[pl.ds(0, HALF)], blo, slo).wait()

        @pl.when(k > 0)
        def _():
            pltpu.make_async_copy(out_v, out_hbm.at[r - 1], so).wait()

        for c8 in range(2 * N_ICH):
            jc = c8 % N_ICH
            par = c8 % 2
            if c8 < 2 * N_ICH - 1:
                njc = (c8 + 1) % N_ICH
                pltpu.async_copy(
                    xt_hbm.at[f].at[pl.ds(njc * ICH, ICH)],
                    ibufs[1 - par], isems[1 - par])
            else:

                @pl.when(k < ROWS_PER_W - 1)
                def _():
                    fn, _en = fe(r + 1)
                    pltpu.async_copy(
                        xt_hbm.at[fn].at[pl.ds(0, ICH)], ibufs[0], isems[0])

            if c8 == N_ICH:
                @pl.when(k < ROWS_PER_W - 1)
                def _():
                    fn, en = fe(r + 1)
                    pltpu.async_copy(
                        tab_hbm.at[fn, en].at[pl.ds(0, HALF)], blo, slo)

                pltpu.make_async_copy(row.at[pl.ds(HALF, HIGH)],
                                      bhi, shi).wait()

            idx_v = ibufs[par]
            pltpu.make_async_copy(xt_hbm.at[f].at[pl.ds(jc * ICH, ICH)],
                                  idx_v, isems[par]).wait()

            if c8 < N_ICH:

                @plsc.parallel_loop(0, ICH // 16, unroll=8)
                def _groups(i):
                    g = idx_v[pl.ds(i * 16, 16)]
                    m = g < HALF
                    out_v[pl.ds(jc * ICH + i * 16, 16)] = (
                        plsc.load_gather(blo, [g], mask=m))

            else:

                @plsc.parallel_loop(0, ICH // 16, unroll=8)
                def _groups(i):
                    g = idx_v[pl.ds(i * 16, 16)]
                    m = g >= HALF
                    gh = plsc.load_gather(bhi, [g - HALF], mask=m)
                    plsc.store_scatter(
                        out_v, [lanes + (jc * ICH + i * 16)], gh, mask=m)

        pltpu.async_copy(out_v, out_hbm.at[r], so)
        return 0

    lax.fori_loop(0, ROWS_PER_W, _task, 0)
    r_last = w * ROWS_PER_W + ROWS_PER_W - 1
    pltpu.make_async_copy(out_v, out_hbm.at[r_last], so).wait()

    @pl.when(w < N_CONT)
    def _cont():
        def _cchunk(j, _):
            pltpu.sync_copy(xt_hbm.at[N_CAT + w].at[pl.ds(j * ICH, ICH)], ia)

            @plsc.parallel_loop(0, ICH // 16, unroll=8)
            def _cgroups(i):
                out_v[pl.ds(j * ICH + i * 16, 16)] = (
                    ia[pl.ds(i * 16, 16)].astype(jnp.float32))

            return 0

        lax.fori_loop(0, N_ICH, _cchunk, 0)
        pltpu.sync_copy(out_v, out_hbm.at[N_ROWS + w])


def kernel(x, tables):
    tab_t = jnp.transpose(tables, (0, 2, 1))
    xt = jnp.transpose(x)
    out_t = _emb_kernel(tab_t, xt)
    return jnp.transpose(out_t)


# speedup vs baseline: 1.0059x; 1.0058x over previous
import functools

import jax
import jax.numpy as jnp
from jax import lax
from jax.experimental import pallas as pl
from jax.experimental.pallas import tpu as pltpu
from jax.experimental.pallas import tpu_sc as plsc

BATCH = 16384
N_CAT = 26
N_CONT = 26
VOCAB = 100000
EMB_DIM = 16
OUT_D = N_CAT * EMB_DIM + N_CONT

NW = 32
N_ROWS = N_CAT * EMB_DIM
ROWS_PER_W = N_ROWS // NW
HALF = 49920
HIGH = VOCAB - HALF
ICH = 4096
N_ICH = BATCH // ICH

_mesh = plsc.VectorSubcoreMesh(core_axis_name="c", subcore_axis_name="s")


@functools.partial(
    pl.kernel,
    mesh=_mesh,
    out_type=jax.ShapeDtypeStruct((OUT_D, BATCH), jnp.float32),
    scratch_types=[
        pltpu.VMEM((HALF,), jnp.float32),
        pltpu.VMEM((HIGH,), jnp.float32),
        pltpu.VMEM((ICH,), jnp.int32),
        pltpu.VMEM((ICH,), jnp.int32),
        pltpu.VMEM((BATCH,), jnp.float32),
        pltpu.SemaphoreType.DMA,
        pltpu.SemaphoreType.DMA,
        pltpu.SemaphoreType.DMA,
        pltpu.SemaphoreType.DMA,
    ],
    compiler_params=pltpu.CompilerParams(needs_layout_passes=False),
)
def _emb_kernel(tab_hbm, xt_hbm, out_hbm, blo, bhi, ia, ib, out_v,
                slo, shi, sa, sb):
    w = lax.axis_index("s") * 2 + lax.axis_index("c")
    ibufs = (ia, ib)
    isems = (sa, sb)
    lanes = lax.iota(jnp.int32, 16)

    def fe(rr):
        return rr // EMB_DIM, rr % EMB_DIM

    r0 = w * ROWS_PER_W
    f0, e0 = fe(r0)
    pltpu.async_copy(tab_hbm.at[f0, e0].at[pl.ds(0, HALF)], blo, slo)
    pltpu.async_copy(xt_hbm.at[f0].at[pl.ds(0, ICH)], ia, sa)

    def _task(k, _):
        r = w * ROWS_PER_W + k
        f, e = fe(r)
        row = tab_hbm.at[f, e]
        pltpu.async_copy(row.at[pl.ds(HALF, HIGH)], bhi, shi)
        pltpu.make_async_copy(row.at[pl.ds(0, HALF)], blo, slo).wait()

        for c8 in range(2 * N_ICH):
            jc = c8 % N_ICH
            par = c8 % 2
            if c8 < 2 * N_ICH - 1:
                njc = (c8 + 1) % N_ICH
                pltpu.async_copy(
                    xt_hbm.at[f].at[pl.ds(njc * ICH, ICH)],
                    ibufs[1 - par], isems[1 - par])
            else:

                @pl.when(k < ROWS_PER_W - 1)
                def _():
                    fn, _en = fe(r + 1)
                    pltpu.async_copy(
                        xt_hbm.at[fn].at[pl.ds(0, ICH)], ibufs[0], isems[0])

            if c8 == N_ICH:
                @pl.when(k < ROWS_PER_W - 1)
                def _():
                    fn, en = fe(r + 1)
                    pltpu.async_copy(
                        tab_hbm.at[fn, en].at[pl.ds(0, HALF)], blo, slo)

                pltpu.make_async_copy(row.at[pl.ds(HALF, HIGH)],
                                      bhi, shi).wait()

            idx_v = ibufs[par]
            pltpu.make_async_copy(xt_hbm.at[f].at[pl.ds(jc * ICH, ICH)],
                                  idx_v, isems[par]).wait()

            if c8 < N_ICH:

                @plsc.parallel_loop(0, ICH // 16, unroll=8)
                def _groups(i):
                    g = idx_v[pl.ds(i * 16, 16)]
                    m = g < HALF
                    out_v[pl.ds(jc * ICH + i * 16, 16)] = (
                        plsc.load_gather(blo, [g], mask=m))

            else:

                @plsc.parallel_loop(0, ICH // 16, unroll=8)
                def _groups(i):
                    g = idx_v[pl.ds(i * 16, 16)]
                    m = g >= HALF
                    gh = plsc.load_gather(bhi, [g - HALF], mask=m)
                    plsc.store_scatter(
                        out_v, [lanes + (jc * ICH + i * 16)], gh, mask=m)

        pltpu.sync_copy(out_v, out_hbm.at[r])
        return 0

    lax.fori_loop(0, ROWS_PER_W, _task, 0)

    @pl.when(w < N_CONT)
    def _cont():
        def _cchunk(j, _):
            pltpu.sync_copy(xt_hbm.at[N_CAT + w].at[pl.ds(j * ICH, ICH)], ia)

            @plsc.parallel_loop(0, ICH // 16, unroll=8)
            def _cgroups(i):
                out_v[pl.ds(j * ICH + i * 16, 16)] = (
                    ia[pl.ds(i * 16, 16)].astype(jnp.float32))

            return 0

        lax.fori_loop(0, N_ICH, _cchunk, 0)
        pltpu.sync_copy(out_v, out_hbm.at[N_ROWS + w])


def kernel(x, tables):
    tab_t = jnp.transpose(tables, (0, 2, 1))
    xt = jnp.transpose(x)
    out_t = _emb_kernel(tab_t, xt)
    return jnp.transpose(out_t)
